# baseline (speedup 1.0000x reference)
import jax
import jax.numpy as jnp
from jax.experimental import pallas as pl
from jax.experimental.pallas import tpu as pltpu

B, N = 8, 2810
IN, HID, LAT, OUT = 256, 128, 64, 128
KDIM = N * LAT
KBLK = 35968
KSTEPS = KDIM // KBLK
RCH = 512
ADJ_CH = 704
NADJ = (N + ADJ_CH - 1) // ADJ_CH


def _gcn_body(x_ref, adj_hbm, w1_ref, w2_ref, out_ref,
              t2_ref, adj_ref, sems):
    j = pl.program_id(0)

    @pl.when(j == 0)
    def _start_adj():
        for c in range(NADJ):
            r0 = c * ADJ_CH
            cr = min(ADJ_CH, N - r0)
            pltpu.make_async_copy(
                adj_hbm.at[r0:r0 + cr, :],
                adj_ref.at[r0:r0 + cr, :],
                sems.at[c],
            ).start()

    @pl.when(j < B // 2)
    def _hop1_pair():
        w1 = w1_ref[...]
        w2 = w2_ref[...]
        tp = jnp.concatenate(
            [jnp.dot(x_ref[0], w1, preferred_element_type=jnp.float32),
             jnp.dot(x_ref[1], w1, preferred_element_type=jnp.float32)],
            axis=1).astype(jnp.bfloat16)
        for c in range(NADJ):
            r0 = c * ADJ_CH
            cr = min(ADJ_CH, N - r0)

            @pl.when(j == 0)
            def _w():
                pltpu.make_async_copy(
                    adj_hbm.at[r0:r0 + cr, :],
                    adj_ref.at[r0:r0 + cr, :],
                    sems.at[c],
                ).wait()

            a_bf = adj_ref[r0:r0 + cr, :].astype(jnp.bfloat16)
            h1_r = jnp.maximum(
                jnp.dot(a_bf, tp, preferred_element_type=jnp.float32), 0.0)
            t2_pair = jnp.concatenate(
                [jnp.dot(h1_r[:, 0:HID], w2,
                         preferred_element_type=jnp.float32),
                 jnp.dot(h1_r[:, HID:2 * HID], w2,
                         preferred_element_type=jnp.float32)],
                axis=1).astype(jnp.bfloat16)
            t2_ref[j, r0:r0 + cr, :] = t2_pair

    @pl.when(j == B // 2)
    def _hop2():
        t2 = jnp.concatenate(
            [t2_ref[p] for p in range(B // 2)], axis=1)
        for r0 in range(0, N, RCH):
            cr = min(RCH, N - r0)
            a_bf = adj_ref[r0:r0 + cr, :].astype(jnp.bfloat16)
            h2_r = jnp.maximum(
                jnp.dot(a_bf, t2, preferred_element_type=jnp.float32), 0.0
            ).astype(jnp.bfloat16)
            for bb in range(B):
                out_ref[bb, r0:r0 + cr, :] = h2_r[:, bb * LAT:(bb + 1) * LAT]


FC_CL = 14080
_FC_CHUNKS = []
_o = 0
while _o < KDIM:
    _FC_CHUNKS.append((_o, min(FC_CL, KDIM - _o)))
    _o += FC_CL
NFC = len(_FC_CHUNKS)


def _fc2_body(flat_ref, wm_hbm, wv_hbm, bm_ref, bv_ref, mean_ref, lv_ref,
              wm0, wm1, wv0, wv1, sems):
    wmb = (wm0, wm1)
    wvb = (wv0, wv1)

    def start(c):
        o0, cl = _FC_CHUNKS[c]
        s = c % 2
        pltpu.make_async_copy(wm_hbm.at[o0:o0 + cl, :],
                              wmb[s].at[0:cl, :], sems.at[0, s]).start()
        pltpu.make_async_copy(wv_hbm.at[o0:o0 + cl, :],
                              wvb[s].at[0:cl, :], sems.at[1, s]).start()

    def wait(c):
        o0, cl = _FC_CHUNKS[c]
        s = c % 2
        pltpu.make_async_copy(wm_hbm.at[o0:o0 + cl, :],
                              wmb[s].at[0:cl, :], sems.at[0, s]).wait()
        pltpu.make_async_copy(wv_hbm.at[o0:o0 + cl, :],
                              wvb[s].at[0:cl, :], sems.at[1, s]).wait()

    start(0)
    pm = bm_ref[...]
    pv = bv_ref[...]
    for c in range(NFC):
        if c + 1 < NFC:
            start(c + 1)
        wait(c)
        o0, cl = _FC_CHUNKS[c]
        s = c % 2
        f = flat_ref[:, o0:o0 + cl].astype(jnp.float32)
        pm = pm + jnp.dot(f, wmb[s][0:cl, :],
                          preferred_element_type=jnp.float32)
        pv = pv + jnp.dot(f, wvb[s][0:cl, :],
                          preferred_element_type=jnp.float32)
    mean_ref[...] = pm
    lv_ref[...] = pv


def _fc_call(flat, Wm, Wv, bm, bv):
    return pl.pallas_call(
        _fc2_body,
        grid=(1,),
        in_specs=[
            pl.BlockSpec((B, KDIM), lambda i: (0, 0)),
            pl.BlockSpec(memory_space=pl.ANY),
            pl.BlockSpec(memory_space=pl.ANY),
            pl.BlockSpec((1, OUT), lambda i: (0, 0)),
            pl.BlockSpec((1, OUT), lambda i: (0, 0)),
        ],
        out_specs=[
            pl.BlockSpec((B, OUT), lambda i: (0, 0)),
            pl.BlockSpec((B, OUT), lambda i: (0, 0)),
        ],
        out_shape=[
            jax.ShapeDtypeStruct((B, OUT), jnp.float32),
            jax.ShapeDtypeStruct((B, OUT), jnp.float32),
        ],
        scratch_shapes=[
            pltpu.VMEM((FC_CL, OUT), jnp.float32),
            pltpu.VMEM((FC_CL, OUT), jnp.float32),
            pltpu.VMEM((FC_CL, OUT), jnp.float32),
            pltpu.VMEM((FC_CL, OUT), jnp.float32),
            pltpu.SemaphoreType.DMA((2, 2)),
        ],
        compiler_params=pltpu.CompilerParams(
            vmem_limit_bytes=60 * 1024 * 1024,
        ),
    )(flat, Wm, Wv, bm, bv)


@jax.jit
def kernel(x, adj, W1, W2, FCm_W, FCm_b, FCv_W, FCv_b):
    h2t = pl.pallas_call(
        _gcn_body,
        grid=(B // 2 + 1,),
        in_specs=[
            pl.BlockSpec((2, N, IN),
                         lambda i: (jnp.minimum(i, B // 2 - 1), 0, 0)),
            pl.BlockSpec(memory_space=pl.ANY),
            pl.BlockSpec((IN, HID), lambda i: (0, 0)),
            pl.BlockSpec((HID, LAT), lambda i: (0, 0)),
        ],
        out_specs=pl.BlockSpec((B, N, LAT), lambda i: (0, 0, 0)),
        out_shape=jax.ShapeDtypeStruct((B, N, LAT), jnp.bfloat16),
        scratch_shapes=[
            pltpu.VMEM((B // 2, N, 2 * LAT), jnp.bfloat16),
            pltpu.VMEM((N, N), jnp.float32),
            pltpu.SemaphoreType.DMA((NADJ,)),
        ],
        compiler_params=pltpu.CompilerParams(
            vmem_limit_bytes=62 * 1024 * 1024,
        ),
    )(x, adj, W1, W2)

    flat = h2t.reshape(B, KDIM)
    mean, log_var = _fc_call(flat, FCm_W, FCv_W,
                             FCm_b.reshape(1, OUT), FCv_b.reshape(1, OUT))
    return (mean, log_var)


# speedup vs baseline: 1.0426x; 1.0426x over previous
import jax
import jax.numpy as jnp
from jax.experimental import pallas as pl
from jax.experimental.pallas import tpu as pltpu

B, N = 8, 2810
IN, HID, LAT, OUT = 256, 128, 64, 128
KDIM = N * LAT
KBLK = 35968
KSTEPS = KDIM // KBLK
RCH = 512
ADJ_CH = 704
NADJ = (N + ADJ_CH - 1) // ADJ_CH


def _gcn_body(x_ref, adj_hbm, w1_ref, w2_ref, out_ref,
              t_ref, t2_ref, adj_ref, sems):
    i = pl.program_id(0)

    @pl.when(i == 0)
    def _start_adj():
        for c in range(NADJ):
            r0 = c * ADJ_CH
            cr = min(ADJ_CH, N - r0)
            pltpu.make_async_copy(
                adj_hbm.at[r0:r0 + cr, :],
                adj_ref.at[r0:r0 + cr, :],
                sems.at[c],
            ).start()

    @pl.when(i < B)
    def _tphase():
        t = jnp.dot(x_ref[0], w1_ref[...], preferred_element_type=jnp.float32)
        for bb in range(B):
            @pl.when(i == bb)
            def _store():
                t_ref[:, bb * HID:(bb + 1) * HID] = t.astype(jnp.bfloat16)

    @pl.when(i == B)
    def _hops():
        for c in range(NADJ):
            r0 = c * ADJ_CH
            cr = min(ADJ_CH, N - r0)
            pltpu.make_async_copy(
                adj_hbm.at[r0:r0 + cr, :],
                adj_ref.at[r0:r0 + cr, :],
                sems.at[c],
            ).wait()
        w2 = w2_ref[...]
        for r0 in range(0, N, RCH):
            cr = min(RCH, N - r0)
            a_bf = adj_ref[r0:r0 + cr, :].astype(jnp.bfloat16)
            h1_r = jnp.maximum(
                jnp.dot(a_bf, t_ref[...],
                        preferred_element_type=jnp.float32), 0.0)
            for bb in range(B):
                t2_ref[r0:r0 + cr, bb * LAT:(bb + 1) * LAT] = jnp.dot(
                    h1_r[:, bb * HID:(bb + 1) * HID], w2,
                    preferred_element_type=jnp.float32).astype(jnp.bfloat16)
        for r0 in range(0, N, RCH):
            cr = min(RCH, N - r0)
            a_bf = adj_ref[r0:r0 + cr, :].astype(jnp.bfloat16)
            h2_r = jnp.maximum(
                jnp.dot(a_bf, t2_ref[...],
                        preferred_element_type=jnp.float32), 0.0
            ).astype(jnp.bfloat16)
            for bb in range(B):
                out_ref[bb, r0:r0 + cr, :] = h2_r[:, bb * LAT:(bb + 1) * LAT]


FC_CL = 14080
_FC_CHUNKS = []
_o = 0
while _o < KDIM:
    _FC_CHUNKS.append((_o, min(FC_CL, KDIM - _o)))
    _o += FC_CL
NFC = len(_FC_CHUNKS)


def _fc2_body(flat_ref, wm_hbm, wv_hbm, bm_ref, bv_ref, mean_ref, lv_ref,
              wm0, wm1, wv0, wv1, sems):
    wmb = (wm0, wm1)
    wvb = (wv0, wv1)

    def start(c):
        o0, cl = _FC_CHUNKS[c]
        s = c % 2
        pltpu.make_async_copy(wm_hbm.at[o0:o0 + cl, :],
                              wmb[s].at[0:cl, :], sems.at[0, s]).start()
        pltpu.make_async_copy(wv_hbm.at[o0:o0 + cl, :],
                              wvb[s].at[0:cl, :], sems.at[1, s]).start()

    def wait(c):
        o0, cl = _FC_CHUNKS[c]
        s = c % 2
        pltpu.make_async_copy(wm_hbm.at[o0:o0 + cl, :],
                              wmb[s].at[0:cl, :], sems.at[0, s]).wait()
        pltpu.make_async_copy(wv_hbm.at[o0:o0 + cl, :],
                              wvb[s].at[0:cl, :], sems.at[1, s]).wait()

    start(0)
    pm = bm_ref[...]
    pv = bv_ref[...]
    for c in range(NFC):
        if c + 1 < NFC:
            start(c + 1)
        wait(c)
        o0, cl = _FC_CHUNKS[c]
        s = c % 2
        f = flat_ref[:, o0:o0 + cl].astype(jnp.float32)
        pm = pm + jnp.dot(f, wmb[s][0:cl, :],
                          preferred_element_type=jnp.float32)
        pv = pv + jnp.dot(f, wvb[s][0:cl, :],
                          preferred_element_type=jnp.float32)
    mean_ref[...] = pm
    lv_ref[...] = pv


def _fc_call(flat, Wm, Wv, bm, bv):
    return pl.pallas_call(
        _fc2_body,
        grid=(1,),
        in_specs=[
            pl.BlockSpec((B, KDIM), lambda i: (0, 0)),
            pl.BlockSpec(memory_space=pl.ANY),
            pl.BlockSpec(memory_space=pl.ANY),
            pl.BlockSpec((1, OUT), lambda i: (0, 0)),
            pl.BlockSpec((1, OUT), lambda i: (0, 0)),
        ],
        out_specs=[
            pl.BlockSpec((B, OUT), lambda i: (0, 0)),
            pl.BlockSpec((B, OUT), lambda i: (0, 0)),
        ],
        out_shape=[
            jax.ShapeDtypeStruct((B, OUT), jnp.float32),
            jax.ShapeDtypeStruct((B, OUT), jnp.float32),
        ],
        scratch_shapes=[
            pltpu.VMEM((FC_CL, OUT), jnp.float32),
            pltpu.VMEM((FC_CL, OUT), jnp.float32),
            pltpu.VMEM((FC_CL, OUT), jnp.float32),
            pltpu.VMEM((FC_CL, OUT), jnp.float32),
            pltpu.SemaphoreType.DMA((2, 2)),
        ],
        compiler_params=pltpu.CompilerParams(
            vmem_limit_bytes=60 * 1024 * 1024,
        ),
    )(flat, Wm, Wv, bm, bv)


@jax.jit
def kernel(x, adj, W1, W2, FCm_W, FCm_b, FCv_W, FCv_b):
    h2t = pl.pallas_call(
        _gcn_body,
        grid=(B + 1,),
        in_specs=[
            pl.BlockSpec((1, N, IN), lambda i: (jnp.minimum(i, B - 1), 0, 0)),
            pl.BlockSpec(memory_space=pl.ANY),
            pl.BlockSpec((IN, HID), lambda i: (0, 0)),
            pl.BlockSpec((HID, LAT), lambda i: (0, 0)),
        ],
        out_specs=pl.BlockSpec((B, N, LAT), lambda i: (0, 0, 0)),
        out_shape=jax.ShapeDtypeStruct((B, N, LAT), jnp.bfloat16),
        scratch_shapes=[
            pltpu.VMEM((N, B * HID), jnp.bfloat16),
            pltpu.VMEM((N, B * LAT), jnp.bfloat16),
            pltpu.VMEM((N, N), jnp.float32),
            pltpu.SemaphoreType.DMA((NADJ,)),
        ],
        compiler_params=pltpu.CompilerParams(
            vmem_limit_bytes=62 * 1024 * 1024,
        ),
    )(x, adj, W1, W2)

    flat = h2t.reshape(B, KDIM)
    mean, log_var = _fc_call(flat, FCm_W, FCv_W,
                             FCm_b.reshape(1, OUT), FCv_b.reshape(1, OUT))
    return (mean, log_var)
